# Initial kernel scaffold; baseline (speedup 1.0000x reference)
import jax
import jax.numpy as jnp
from jax.experimental import pallas as pl


def kernel(x, edge_index, params):
    raise NotImplementedError("write your pallas kernel here")


# speedup vs baseline: 2.2762x; 2.2762x over previous
import functools

import jax
import jax.numpy as jnp
from jax.experimental import pallas as pl
from jax.experimental.pallas import tpu as pltpu

N_NODES = 10000
K_VOL = 27
E_PER_K = 5000


def _mm_body(g_ref, w_ref, o_ref):
    o_ref[...] = jnp.dot(g_ref[0], w_ref[0],
                         preferred_element_type=jnp.float32)[None]


def _batched_mm(g, w):
    k, e, cin = g.shape
    cout = w.shape[2]
    return pl.pallas_call(
        _mm_body,
        grid=(k,),
        in_specs=[
            pl.BlockSpec((1, e, cin), lambda i: (i, 0, 0)),
            pl.BlockSpec((1, cin, cout), lambda i: (i, 0, 0)),
        ],
        out_specs=pl.BlockSpec((1, e, cout), lambda i: (i, 0, 0)),
        out_shape=jax.ShapeDtypeStruct((k, e, cout), jnp.float32),
    )(g, w)


def _bn_body(relu, has_res, *refs):
    if has_res:
        s_ref, g_ref, b_ref, r_ref, o_ref = refs
    else:
        s_ref, g_ref, b_ref, o_ref = refs
    x = s_ref[...]
    m = jnp.mean(x, axis=0, keepdims=True)
    v = jnp.mean((x - m) * (x - m), axis=0, keepdims=True)
    y = (x - m) * jax.lax.rsqrt(v + 1e-5) * g_ref[...] + b_ref[...]
    if has_res:
        y = y + r_ref[...]
    if relu:
        y = jnp.maximum(y, 0.0)
    o_ref[...] = y


def _bn_act(s, g, b, residual=None, relu=True):
    n, c = s.shape
    args = [s, g.reshape(1, c), b.reshape(1, c)]
    if residual is not None:
        args.append(residual)
    return pl.pallas_call(
        functools.partial(_bn_body, relu, residual is not None),
        out_shape=jax.ShapeDtypeStruct((n, c), jnp.float32),
    )(*args)


def _final_body(x_ref, w_ref, b_ref, o_ref):
    y = jnp.dot(jnp.maximum(x_ref[...], 0.0), w_ref[...],
                preferred_element_type=jnp.float32) + b_ref[...]
    nrm = jnp.sqrt(jnp.sum(y * y, axis=1, keepdims=True))
    o_ref[...] = y / (nrm + 1e-8)


def _final(x, w, b):
    n = x.shape[0]
    cout = w.shape[1]
    return pl.pallas_call(
        _final_body,
        out_shape=jax.ShapeDtypeStruct((n, cout), jnp.float32),
    )(x, w.astype(jnp.float32), b.reshape(1, cout))


def _sconv(x, w, src, dst_flat):
    cin = x.shape[1]
    if w.shape[1] != cin:
        w = w[:, :cin, :]
    g = x[src.reshape(-1)].reshape(K_VOL, E_PER_K, cin)
    y = _batched_mm(g, w)
    return jax.ops.segment_sum(y.reshape(K_VOL * E_PER_K, -1), dst_flat,
                               num_segments=N_NODES)


def _block(x, p, src, dst_flat):
    o = _bn_act(_sconv(x, p["w1"], src, dst_flat), p["bn1"]["g"], p["bn1"]["b"],
                relu=True)
    o = _bn_act(_sconv(o, p["w2"], src, dst_flat), p["bn2"]["g"], p["bn2"]["b"],
                residual=x, relu=True)
    return o


def kernel(x, edge_index, params):
    src_f = edge_index[:, 0, :]
    dst_f = edge_index[:, 1, :].reshape(-1)
    src_t = edge_index[:, 1, :]
    dst_t = edge_index[:, 0, :].reshape(-1)

    p = params
    xp = jnp.pad(x, ((0, 0), (0, 13)))
    w1 = jnp.pad(p["conv1"], ((0, 0), (0, 13), (0, 0)))

    def conv_bn(h, w, nrm, transposed=False):
        s = src_t if transposed else src_f
        d = dst_t if transposed else dst_f
        return _bn_act(_sconv(h, w, s, d), nrm["g"], nrm["b"], relu=False)

    def blk(h, bp):
        return _block(h, bp, src_f, dst_f)

    s1 = blk(conv_bn(xp, w1, p["norm1"]), p["block1"])
    s2 = blk(conv_bn(s1, p["conv2"], p["norm2"]), p["block2"])
    s4 = blk(conv_bn(s2, p["conv3"], p["norm3"]), p["block3"])
    s8 = blk(conv_bn(s4, p["conv4"], p["norm4"]), p["block4"])

    out = blk(conv_bn(s8, p["conv4_tr"], p["norm4_tr"], True), p["block4_tr"])
    out = jnp.concatenate([out, s4], axis=1)
    out = blk(conv_bn(out, p["conv3_tr"], p["norm3_tr"], True), p["block3_tr"])
    out = jnp.concatenate([out, s2], axis=1)
    out = blk(conv_bn(out, p["conv2_tr"], p["norm2_tr"], True), p["block2_tr"])
    out = jnp.concatenate([out, s1], axis=1)

    src_t_flat = src_t.reshape(-1)
    g = out[src_t_flat].reshape(K_VOL, E_PER_K, out.shape[1])
    y = _batched_mm(g, p["conv1_tr"])
    out = jax.ops.segment_sum(y.reshape(K_VOL * E_PER_K, -1), dst_t,
                              num_segments=N_NODES)
    return _final(out, p["final_w"], p["final_b"])


# speedup vs baseline: 3.4174x; 1.5013x over previous
import functools

import jax
import jax.numpy as jnp
from jax import lax
from jax.experimental import pallas as pl
from jax.experimental.pallas import tpu as pltpu
from jax.experimental.pallas import tpu_sc as plsc

N_NODES = 10000
K_VOL = 27
E_PER_K = 5000

_NC = 2
_NS = 16
_NW = _NC * _NS
E_PAD_K = 5120
E_PAD = K_VOL * E_PAD_K
N_PAD = 10240
_RG = 120


def _ceil128(c):
    return ((c + 127) // 128) * 128


@functools.partial(jax.jit, static_argnames=("d",))
def _sc_gather(table, idx, d):
    per_w = E_PAD // _NW
    chunks = per_w // _RG
    mesh = plsc.VectorSubcoreMesh(core_axis_name="c", subcore_axis_name="s")

    @functools.partial(
        pl.kernel,
        mesh=mesh,
        out_type=jax.ShapeDtypeStruct((E_PAD, d), jnp.float32),
        scratch_types=[
            pltpu.VMEM((_RG,), jnp.int32),
            pltpu.VMEM((_RG, d), jnp.float32),
            pltpu.SemaphoreType.DMA,
        ],
    )
    def k(table_hbm, idx_hbm, out_hbm, idx_v, rows_v, sem):
        wid = lax.axis_index("s") * _NC + lax.axis_index("c")
        base = wid * per_w

        def body(i, carry):
            off = base + i * _RG
            pltpu.sync_copy(idx_hbm.at[pl.ds(off, _RG)], idx_v)
            pltpu.async_copy(table_hbm.at[idx_v], rows_v, sem).wait()
            pltpu.sync_copy(rows_v, out_hbm.at[pl.ds(off, _RG)])
            return carry

        lax.fori_loop(0, chunks, body, 0)

    return k(table, idx)


def _mm_body(g_ref, w_ref, o_ref):
    o_ref[...] = jnp.dot(g_ref[0], w_ref[0],
                         preferred_element_type=jnp.float32)[None]


def _batched_mm(g, w):
    k, e, cin = g.shape
    cout = w.shape[2]
    return pl.pallas_call(
        _mm_body,
        grid=(k,),
        in_specs=[
            pl.BlockSpec((1, e, cin), lambda i: (i, 0, 0)),
            pl.BlockSpec((1, cin, cout), lambda i: (i, 0, 0)),
        ],
        out_specs=pl.BlockSpec((1, e, cout), lambda i: (i, 0, 0)),
        out_shape=jax.ShapeDtypeStruct((k, e, cout), jnp.float32),
    )(g, w)


def _bn_body(relu, has_res, cp, *refs):
    if has_res:
        s_ref, g_ref, b_ref, r_ref, o_ref = refs
    else:
        s_ref, g_ref, b_ref, o_ref = refs
    x = s_ref[...]
    n, c = x.shape
    m = jnp.mean(x, axis=0, keepdims=True)
    v = jnp.mean((x - m) * (x - m), axis=0, keepdims=True)
    y = (x - m) * jax.lax.rsqrt(v + 1e-5) * g_ref[...] + b_ref[...]
    if cp > c:
        y = jnp.concatenate([y, jnp.zeros((n, cp - c), y.dtype)], axis=1)
    if has_res:
        y = y + r_ref[...]
    if relu:
        y = jnp.maximum(y, 0.0)
    o_ref[...] = y


def _bn_act(s, g, b, residual=None, relu=True):
    n, c = s.shape
    cp = _ceil128(c)
    args = [s, g.reshape(1, c), b.reshape(1, c)]
    if residual is not None:
        args.append(residual)
    return pl.pallas_call(
        functools.partial(_bn_body, relu, residual is not None, cp),
        out_shape=jax.ShapeDtypeStruct((n, cp), jnp.float32),
    )(*args)


def _final_body(x_ref, w_ref, b_ref, o_ref):
    y = jnp.dot(jnp.maximum(x_ref[...], 0.0), w_ref[...],
                preferred_element_type=jnp.float32) + b_ref[...]
    nrm = jnp.sqrt(jnp.sum(y * y, axis=1, keepdims=True))
    o_ref[...] = y / (nrm + 1e-8)


def _final(x, w, b):
    n = x.shape[0]
    cout = w.shape[1]
    return pl.pallas_call(
        _final_body,
        out_shape=jax.ShapeDtypeStruct((n, cout), jnp.float32),
    )(x, w.astype(jnp.float32), b.reshape(1, cout))


def _pad_w(w, segs):
    parts, o = [], 0
    for real, padw in segs:
        parts.append(w[:, o:o + real, :])
        if padw > real:
            parts.append(jnp.zeros((K_VOL, padw - real, w.shape[2]), w.dtype))
        o += real
    if len(parts) == 1:
        return parts[0]
    return jnp.concatenate(parts, axis=1)


def _sconv(x_p, w_p, src_flat, dst_flat):
    cp = x_p.shape[1]
    g = _sc_gather(x_p, src_flat, cp).reshape(K_VOL, E_PAD_K, cp)
    y = _batched_mm(g, w_p)
    s = jax.ops.segment_sum(y.reshape(E_PAD, -1), dst_flat, num_segments=N_PAD)
    return s[:N_NODES]


def _block(x_p, c, p, src, dst):
    cp = x_p.shape[1]
    segs = [(c, cp)]
    o = _bn_act(_sconv(x_p, _pad_w(p["w1"], segs), src, dst),
                p["bn1"]["g"], p["bn1"]["b"], relu=True)
    o = _bn_act(_sconv(o, _pad_w(p["w2"], segs), src, dst),
                p["bn2"]["g"], p["bn2"]["b"], residual=x_p, relu=True)
    return o


def _prep_edges(srcs, dsts):
    src_p = jnp.pad(srcs, ((0, 0), (0, E_PAD_K - E_PER_K)))
    padvals = (N_NODES + (jnp.arange(E_PAD_K - E_PER_K) % (N_PAD - N_NODES)))
    dst_p = jnp.concatenate(
        [dsts, jnp.broadcast_to(padvals.astype(jnp.int32),
                                (K_VOL, E_PAD_K - E_PER_K))], axis=1)
    return src_p.reshape(-1), dst_p.reshape(-1)


def kernel(x, edge_index, params):
    src_f, dst_f = _prep_edges(edge_index[:, 0, :], edge_index[:, 1, :])
    src_t, dst_t = _prep_edges(edge_index[:, 1, :], edge_index[:, 0, :])

    p = params
    xp = jnp.pad(x, ((0, 0), (0, 125)))

    def conv_bn(h_p, segs, w, nrm, transposed=False):
        s = src_t if transposed else src_f
        d = dst_t if transposed else dst_f
        return _bn_act(_sconv(h_p, _pad_w(w, segs), s, d),
                       nrm["g"], nrm["b"], relu=False)

    def blk(h_p, c, bp):
        return _block(h_p, c, bp, src_f, dst_f)

    s1 = blk(conv_bn(xp, [(3, 128)], p["conv1"], p["norm1"]), 32, p["block1"])
    s2 = blk(conv_bn(s1, [(32, 128)], p["conv2"], p["norm2"]), 64, p["block2"])
    s4 = blk(conv_bn(s2, [(64, 128)], p["conv3"], p["norm3"]), 128, p["block3"])
    s8 = blk(conv_bn(s4, [(128, 128)], p["conv4"], p["norm4"]), 256, p["block4"])

    out = blk(conv_bn(s8, [(256, 256)], p["conv4_tr"], p["norm4_tr"], True),
              128, p["block4_tr"])
    out = jnp.concatenate([out, s4], axis=1)
    out = blk(conv_bn(out, [(128, 128), (128, 128)], p["conv3_tr"],
                      p["norm3_tr"], True), 64, p["block3_tr"])
    out = jnp.concatenate([out, s2], axis=1)
    out = blk(conv_bn(out, [(64, 128), (64, 128)], p["conv2_tr"],
                      p["norm2_tr"], True), 64, p["block2_tr"])
    out = jnp.concatenate([out, s1], axis=1)

    out = _sconv(out, _pad_w(p["conv1_tr"], [(64, 128), (32, 128)]),
                 src_t, dst_t)
    return _final(out, p["final_w"], p["final_b"])


# speedup vs baseline: 4.9699x; 1.4543x over previous
import functools

import jax
import jax.numpy as jnp
from jax import lax
from jax.experimental import pallas as pl
from jax.experimental.pallas import tpu as pltpu
from jax.experimental.pallas import tpu_sc as plsc

N_NODES = 10000
K_VOL = 27
E_PER_K = 5000

_NC = 2
_NS = 16
_NW = _NC * _NS
E_PAD_K = 5120
E_PAD = K_VOL * E_PAD_K
N_PAD = 10240
_RG = 120


def _ceil128(c):
    return ((c + 127) // 128) * 128


@functools.partial(jax.jit, static_argnames=("d",))
def _sc_gather(table, idx, d):
    per_w = E_PAD // _NW
    chunks = per_w // _RG
    mesh = plsc.VectorSubcoreMesh(core_axis_name="c", subcore_axis_name="s")

    @functools.partial(
        pl.kernel,
        mesh=mesh,
        out_type=jax.ShapeDtypeStruct((E_PAD, d), jnp.float32),
        scratch_types=[
            pltpu.VMEM((_RG,), jnp.int32),
            pltpu.VMEM((_RG, d), jnp.float32),
            pltpu.SemaphoreType.DMA,
        ],
    )
    def k(table_hbm, idx_hbm, out_hbm, idx_v, rows_v, sem):
        wid = lax.axis_index("s") * _NC + lax.axis_index("c")
        base = wid * per_w

        def body(i, carry):
            off = base + i * _RG
            pltpu.sync_copy(idx_hbm.at[pl.ds(off, _RG)], idx_v)
            pltpu.async_copy(table_hbm.at[idx_v], rows_v, sem).wait()
            pltpu.sync_copy(rows_v, out_hbm.at[pl.ds(off, _RG)])
            return carry

        lax.fori_loop(0, chunks, body, 0)

    return k(table, idx)


@functools.partial(jax.jit, static_argnames=("split_ch",))
def _sc_scatter(y, dst, split_ch):
    cout = 128
    per_sub_rows = N_PAD // _NS
    per_edge = (E_PAD if split_ch else E_PAD // 2) // _NS
    chunks = per_edge // _RG
    zeros = jnp.zeros((N_PAD, cout), jnp.float32)
    mesh = plsc.VectorSubcoreMesh(core_axis_name="c", subcore_axis_name="s")

    @functools.partial(
        pl.kernel,
        mesh=mesh,
        out_type=jax.ShapeDtypeStruct((2, N_PAD, cout), jnp.float32),
        scratch_types=[
            pltpu.VMEM((_RG,), jnp.int32),
            pltpu.VMEM((_RG, cout), jnp.float32),
            pltpu.VMEM_SHARED((N_PAD, cout), jnp.float32),
        ],
    )
    def k(y_hbm, dst_hbm, z_hbm, out_hbm, idx_v, rows_v, acc):
        c = lax.axis_index("c")
        s = lax.axis_index("s")
        row0 = s * per_sub_rows
        pltpu.sync_copy(z_hbm.at[pl.ds(row0, per_sub_rows)],
                        acc.at[pl.ds(row0, per_sub_rows)])
        plsc.subcore_barrier()
        base = s * per_edge if split_ch else c * (E_PAD // 2) + s * per_edge

        def body(i, carry):
            off = base + i * _RG
            pltpu.sync_copy(dst_hbm.at[pl.ds(off, _RG)], idx_v)
            if split_ch:
                pltpu.sync_copy(y_hbm.at[c, pl.ds(off, _RG)], rows_v)
            else:
                pltpu.sync_copy(y_hbm.at[pl.ds(off, _RG)], rows_v)
            pltpu.sync_copy(rows_v, acc.at[idx_v], add=True)
            return carry

        lax.fori_loop(0, chunks, body, 0)
        plsc.subcore_barrier()
        pltpu.sync_copy(acc.at[pl.ds(row0, per_sub_rows)],
                        out_hbm.at[c, pl.ds(row0, per_sub_rows)])

    return k(y, dst, zeros)


def _mm_body(g_ref, w_ref, o_ref):
    o_ref[...] = jnp.dot(g_ref[0], w_ref[0],
                         preferred_element_type=jnp.float32)[None]


def _batched_mm(g, w):
    k, e, cin = g.shape
    cout = w.shape[2]
    if cout < 128:
        w = jnp.concatenate(
            [w, jnp.zeros((k, cin, 128 - cout), w.dtype)], axis=2)
    return pl.pallas_call(
        _mm_body,
        grid=(k,),
        in_specs=[
            pl.BlockSpec((1, e, cin), lambda i: (i, 0, 0)),
            pl.BlockSpec((1, cin, 128), lambda i: (i, 0, 0)),
        ],
        out_specs=pl.BlockSpec((1, e, 128), lambda i: (i, 0, 0)),
        out_shape=jax.ShapeDtypeStruct((k, e, 128), jnp.float32),
    )(g, w)


def _mm_split_body(g_ref, w_ref, o_ref):
    o_ref[...] = jnp.dot(g_ref[0], w_ref[0],
                         preferred_element_type=jnp.float32)[None, None]


def _batched_mm_split(g, w):
    k, e, cin = g.shape
    return pl.pallas_call(
        _mm_split_body,
        grid=(k, 2),
        in_specs=[
            pl.BlockSpec((1, e, cin), lambda i, h: (i, 0, 0)),
            pl.BlockSpec((1, cin, 128), lambda i, h: (i, 0, h)),
        ],
        out_specs=pl.BlockSpec((1, 1, e, 128), lambda i, h: (h, i, 0, 0)),
        out_shape=jax.ShapeDtypeStruct((2, k, e, 128), jnp.float32),
    )(g, w)


def _bn_body(relu, has_res, cp, concat_mode, *refs):
    if has_res:
        s_ref, g_ref, b_ref, r_ref, o_ref = refs
    else:
        s_ref, g_ref, b_ref, o_ref = refs
    if concat_mode:
        x = jnp.concatenate([s_ref[0, :N_NODES, :], s_ref[1, :N_NODES, :]],
                            axis=1)
    else:
        x = s_ref[0, :N_NODES, :] + s_ref[1, :N_NODES, :]
    n = x.shape[0]
    c = g_ref.shape[1]
    x = x[:, :c]
    m = jnp.mean(x, axis=0, keepdims=True)
    v = jnp.mean((x - m) * (x - m), axis=0, keepdims=True)
    y = (x - m) * jax.lax.rsqrt(v + 1e-5) * g_ref[...] + b_ref[...]
    if cp > c:
        y = jnp.concatenate([y, jnp.zeros((n, cp - c), y.dtype)], axis=1)
    if has_res:
        y = y + r_ref[...]
    if relu:
        y = jnp.maximum(y, 0.0)
    o_ref[...] = y


def _bn_act(s_pair, concat_mode, g, b, residual=None, relu=True):
    c = g.shape[0]
    cp = _ceil128(c)
    args = [s_pair, g.reshape(1, c), b.reshape(1, c)]
    if residual is not None:
        args.append(residual)
    return pl.pallas_call(
        functools.partial(_bn_body, relu, residual is not None, cp,
                          concat_mode),
        out_shape=jax.ShapeDtypeStruct((N_NODES, cp), jnp.float32),
    )(*args)


def _final_body(x_ref, w_ref, b_ref, o_ref):
    cin = w_ref.shape[0]
    x = (x_ref[0, :N_NODES, :] + x_ref[1, :N_NODES, :])[:, :cin]
    y = jnp.dot(jnp.maximum(x, 0.0), w_ref[...],
                preferred_element_type=jnp.float32) + b_ref[...]
    nrm = jnp.sqrt(jnp.sum(y * y, axis=1, keepdims=True))
    o_ref[...] = y / (nrm + 1e-8)


def _final(x_pair, w, b):
    cout = w.shape[1]
    return pl.pallas_call(
        _final_body,
        out_shape=jax.ShapeDtypeStruct((N_NODES, cout), jnp.float32),
    )(x_pair, w.astype(jnp.float32), b.reshape(1, cout))


def _pad_w(w, segs):
    parts, o = [], 0
    for real, padw in segs:
        parts.append(w[:, o:o + real, :])
        if padw > real:
            parts.append(jnp.zeros((K_VOL, padw - real, w.shape[2]), w.dtype))
        o += real
    if len(parts) == 1:
        return parts[0]
    return jnp.concatenate(parts, axis=1)


def _sconv(x_p, w_p, src_flat, dst_flat):
    cp = x_p.shape[1]
    cout = w_p.shape[2]
    g = _sc_gather(x_p, src_flat, cp).reshape(K_VOL, E_PAD_K, cp)
    if cout == 256:
        y = _batched_mm_split(g, w_p)
        return _sc_scatter(y.reshape(2, E_PAD, 128), dst_flat, True), True
    y = _batched_mm(g, w_p)
    return _sc_scatter(y.reshape(E_PAD, 128), dst_flat, False), False


def _block(x_p, c, p, src, dst):
    cp = x_p.shape[1]
    segs = [(c, cp)]
    s, cm = _sconv(x_p, _pad_w(p["w1"], segs), src, dst)
    o = _bn_act(s, cm, p["bn1"]["g"], p["bn1"]["b"], relu=True)
    s, cm = _sconv(o, _pad_w(p["w2"], segs), src, dst)
    o = _bn_act(s, cm, p["bn2"]["g"], p["bn2"]["b"], residual=x_p, relu=True)
    return o


def _prep_edges(srcs, dsts):
    src_p = jnp.pad(srcs, ((0, 0), (0, E_PAD_K - E_PER_K)))
    padvals = (N_NODES + (jnp.arange(E_PAD_K - E_PER_K) % (N_PAD - N_NODES)))
    dst_p = jnp.concatenate(
        [dsts, jnp.broadcast_to(padvals.astype(jnp.int32),
                                (K_VOL, E_PAD_K - E_PER_K))], axis=1)
    return src_p.reshape(-1), dst_p.reshape(-1)


def kernel(x, edge_index, params):
    src_f, dst_f = _prep_edges(edge_index[:, 0, :], edge_index[:, 1, :])
    src_t, dst_t = _prep_edges(edge_index[:, 1, :], edge_index[:, 0, :])

    p = params
    xp = jnp.pad(x, ((0, 0), (0, 125)))

    def conv_bn(h_p, segs, w, nrm, transposed=False):
        s = src_t if transposed else src_f
        d = dst_t if transposed else dst_f
        sp, cm = _sconv(h_p, _pad_w(w, segs), s, d)
        return _bn_act(sp, cm, nrm["g"], nrm["b"], relu=False)

    def blk(h_p, c, bp):
        return _block(h_p, c, bp, src_f, dst_f)

    s1 = blk(conv_bn(xp, [(3, 128)], p["conv1"], p["norm1"]), 32, p["block1"])
    s2 = blk(conv_bn(s1, [(32, 128)], p["conv2"], p["norm2"]), 64, p["block2"])
    s4 = blk(conv_bn(s2, [(64, 128)], p["conv3"], p["norm3"]), 128, p["block3"])
    s8 = blk(conv_bn(s4, [(128, 128)], p["conv4"], p["norm4"]), 256, p["block4"])

    out = blk(conv_bn(s8, [(256, 256)], p["conv4_tr"], p["norm4_tr"], True),
              128, p["block4_tr"])
    out = jnp.concatenate([out, s4], axis=1)
    out = blk(conv_bn(out, [(128, 128), (128, 128)], p["conv3_tr"],
                      p["norm3_tr"], True), 64, p["block3_tr"])
    out = jnp.concatenate([out, s2], axis=1)
    out = blk(conv_bn(out, [(64, 128), (64, 128)], p["conv2_tr"],
                      p["norm2_tr"], True), 64, p["block2_tr"])
    out = jnp.concatenate([out, s1], axis=1)

    sp, _ = _sconv(out, _pad_w(p["conv1_tr"], [(64, 128), (32, 128)]),
                   src_t, dst_t)
    return _final(sp, p["final_w"], p["final_b"])


# speedup vs baseline: 5.0303x; 1.0122x over previous
import functools

import jax
import jax.numpy as jnp
from jax import lax
from jax.experimental import pallas as pl
from jax.experimental.pallas import tpu as pltpu
from jax.experimental.pallas import tpu_sc as plsc

N_NODES = 10000
K_VOL = 27
E_PER_K = 5000

_NC = 2
_NS = 16
_NW = _NC * _NS
E_PAD_K = 5120
E_PAD = K_VOL * E_PAD_K
N_PAD = 10240
_RG = 120


def _ceil128(c):
    return ((c + 127) // 128) * 128


@functools.partial(jax.jit, static_argnames=("d",))
def _sc_gather(table, idx, d):
    per_w = E_PAD // _NW
    chunks = per_w // _RG
    pairs = chunks // 2
    mesh = plsc.VectorSubcoreMesh(core_axis_name="c", subcore_axis_name="s")

    @functools.partial(
        pl.kernel,
        mesh=mesh,
        out_type=jax.ShapeDtypeStruct((E_PAD, d), jnp.float32),
        scratch_types=[
            pltpu.VMEM((chunks, _RG), jnp.int32),
            pltpu.VMEM((_RG, d), jnp.float32),
            pltpu.VMEM((_RG, d), jnp.float32),
            pltpu.SemaphoreType.DMA,
            pltpu.SemaphoreType.DMA,
            pltpu.SemaphoreType.DMA,
            pltpu.SemaphoreType.DMA,
        ],
    )
    def k(table_hbm, idx_hbm, out_hbm, idx_v, rows0, rows1, g0, g1, s0, s1):
        wid = lax.axis_index("s") * _NC + lax.axis_index("c")
        base = wid * per_w
        pltpu.sync_copy(idx_hbm.at[wid], idx_v)

        def g_start(i, buf, sem):
            return pltpu.async_copy(table_hbm.at[idx_v.at[i]], buf, sem)

        def g_wait(buf, sem):
            pltpu.make_async_copy(table_hbm.at[idx_v.at[0]], buf, sem).wait()

        def s_start(i, buf, sem):
            return pltpu.async_copy(buf, out_hbm.at[pl.ds(base + i * _RG, _RG)],
                                    sem)

        def s_wait(buf, sem):
            pltpu.make_async_copy(buf, out_hbm.at[pl.ds(base, _RG)], sem).wait()

        g_start(0, rows0, g0)

        def body(j, carry):
            i = 2 * j
            g_wait(rows0, g0)
            g_start(i + 1, rows1, g1)
            s_start(i, rows0, s0)
            g_wait(rows1, g1)
            s_wait(rows0, s0)

            @pl.when(j < pairs - 1)
            def _():
                g_start(i + 2, rows0, g0)

            s_start(i + 1, rows1, s1)
            s_wait(rows1, s1)
            return carry

        lax.fori_loop(0, pairs, body, 0)

    return k(table, idx)


@functools.partial(jax.jit, static_argnames=("split_ch",))
def _sc_scatter(y, dst, split_ch):
    cout = 128
    per_sub_rows = N_PAD // _NS
    per_edge = (E_PAD if split_ch else E_PAD // 2) // _NS
    chunks = per_edge // _RG
    zeros = jnp.zeros((N_PAD, cout), jnp.float32)
    mesh = plsc.VectorSubcoreMesh(core_axis_name="c", subcore_axis_name="s")

    @functools.partial(
        pl.kernel,
        mesh=mesh,
        out_type=jax.ShapeDtypeStruct((2, N_PAD, cout), jnp.float32),
        scratch_types=[
            pltpu.VMEM((_RG,), jnp.int32),
            pltpu.VMEM((_RG, cout), jnp.float32),
            pltpu.VMEM_SHARED((N_PAD, cout), jnp.float32),
        ],
    )
    def k(y_hbm, dst_hbm, z_hbm, out_hbm, idx_v, rows_v, acc):
        c = lax.axis_index("c")
        s = lax.axis_index("s")
        row0 = s * per_sub_rows
        pltpu.sync_copy(z_hbm.at[pl.ds(row0, per_sub_rows)],
                        acc.at[pl.ds(row0, per_sub_rows)])
        plsc.subcore_barrier()
        base = s * per_edge if split_ch else c * (E_PAD // 2) + s * per_edge

        def body(i, carry):
            off = base + i * _RG
            pltpu.sync_copy(dst_hbm.at[pl.ds(off, _RG)], idx_v)
            if split_ch:
                pltpu.sync_copy(y_hbm.at[c, pl.ds(off, _RG)], rows_v)
            else:
                pltpu.sync_copy(y_hbm.at[pl.ds(off, _RG)], rows_v)
            pltpu.sync_copy(rows_v, acc.at[idx_v], add=True)
            return carry

        lax.fori_loop(0, chunks, body, 0)
        plsc.subcore_barrier()
        pltpu.sync_copy(acc.at[pl.ds(row0, per_sub_rows)],
                        out_hbm.at[c, pl.ds(row0, per_sub_rows)])

    return k(y, dst, zeros)


def _mm_body(g_ref, w_ref, o_ref):
    o_ref[...] = jnp.dot(g_ref[0], w_ref[0],
                         preferred_element_type=jnp.float32)[None]


def _batched_mm(g, w):
    k, e, cin = g.shape
    cout = w.shape[2]
    if cout < 128:
        w = jnp.concatenate(
            [w, jnp.zeros((k, cin, 128 - cout), w.dtype)], axis=2)
    return pl.pallas_call(
        _mm_body,
        grid=(k,),
        in_specs=[
            pl.BlockSpec((1, e, cin), lambda i: (i, 0, 0)),
            pl.BlockSpec((1, cin, 128), lambda i: (i, 0, 0)),
        ],
        out_specs=pl.BlockSpec((1, e, 128), lambda i: (i, 0, 0)),
        out_shape=jax.ShapeDtypeStruct((k, e, 128), jnp.float32),
    )(g, w)


def _mm_split_body(g_ref, w_ref, o_ref):
    o_ref[...] = jnp.dot(g_ref[0], w_ref[0],
                         preferred_element_type=jnp.float32)[None, None]


def _batched_mm_split(g, w):
    k, e, cin = g.shape
    return pl.pallas_call(
        _mm_split_body,
        grid=(k, 2),
        in_specs=[
            pl.BlockSpec((1, e, cin), lambda i, h: (i, 0, 0)),
            pl.BlockSpec((1, cin, 128), lambda i, h: (i, 0, h)),
        ],
        out_specs=pl.BlockSpec((1, 1, e, 128), lambda i, h: (h, i, 0, 0)),
        out_shape=jax.ShapeDtypeStruct((2, k, e, 128), jnp.float32),
    )(g, w)


def _bn_body(relu, has_res, cp, concat_mode, *refs):
    if has_res:
        s_ref, g_ref, b_ref, r_ref, o_ref = refs
    else:
        s_ref, g_ref, b_ref, o_ref = refs
    if concat_mode:
        x = jnp.concatenate([s_ref[0, :N_NODES, :], s_ref[1, :N_NODES, :]],
                            axis=1)
    else:
        x = s_ref[0, :N_NODES, :] + s_ref[1, :N_NODES, :]
    n = x.shape[0]
    c = g_ref.shape[1]
    x = x[:, :c]
    m = jnp.mean(x, axis=0, keepdims=True)
    v = jnp.mean((x - m) * (x - m), axis=0, keepdims=True)
    y = (x - m) * jax.lax.rsqrt(v + 1e-5) * g_ref[...] + b_ref[...]
    if cp > c:
        y = jnp.concatenate([y, jnp.zeros((n, cp - c), y.dtype)], axis=1)
    if has_res:
        y = y + r_ref[...]
    if relu:
        y = jnp.maximum(y, 0.0)
    o_ref[...] = y


def _bn_act(s_pair, concat_mode, g, b, residual=None, relu=True):
    c = g.shape[0]
    cp = _ceil128(c)
    args = [s_pair, g.reshape(1, c), b.reshape(1, c)]
    if residual is not None:
        args.append(residual)
    return pl.pallas_call(
        functools.partial(_bn_body, relu, residual is not None, cp,
                          concat_mode),
        out_shape=jax.ShapeDtypeStruct((N_NODES, cp), jnp.float32),
    )(*args)


def _final_body(x_ref, w_ref, b_ref, o_ref):
    cin = w_ref.shape[0]
    x = (x_ref[0, :N_NODES, :] + x_ref[1, :N_NODES, :])[:, :cin]
    y = jnp.dot(jnp.maximum(x, 0.0), w_ref[...],
                preferred_element_type=jnp.float32) + b_ref[...]
    nrm = jnp.sqrt(jnp.sum(y * y, axis=1, keepdims=True))
    o_ref[...] = y / (nrm + 1e-8)


def _final(x_pair, w, b):
    cout = w.shape[1]
    return pl.pallas_call(
        _final_body,
        out_shape=jax.ShapeDtypeStruct((N_NODES, cout), jnp.float32),
    )(x_pair, w.astype(jnp.float32), b.reshape(1, cout))


def _pad_w(w, segs):
    parts, o = [], 0
    for real, padw in segs:
        parts.append(w[:, o:o + real, :])
        if padw > real:
            parts.append(jnp.zeros((K_VOL, padw - real, w.shape[2]), w.dtype))
        o += real
    if len(parts) == 1:
        return parts[0]
    return jnp.concatenate(parts, axis=1)


def _sconv(x_p, w_p, src_flat, dst_flat):
    cp = x_p.shape[1]
    cout = w_p.shape[2]
    g = _sc_gather(x_p, src_flat.reshape(_NW, -1, _RG), cp).reshape(
        K_VOL, E_PAD_K, cp)
    if cout == 256:
        y = _batched_mm_split(g, w_p)
        return _sc_scatter(y.reshape(2, E_PAD, 128), dst_flat, True), True
    y = _batched_mm(g, w_p)
    return _sc_scatter(y.reshape(E_PAD, 128), dst_flat, False), False


def _block(x_p, c, p, src, dst):
    cp = x_p.shape[1]
    segs = [(c, cp)]
    s, cm = _sconv(x_p, _pad_w(p["w1"], segs), src, dst)
    o = _bn_act(s, cm, p["bn1"]["g"], p["bn1"]["b"], relu=True)
    s, cm = _sconv(o, _pad_w(p["w2"], segs), src, dst)
    o = _bn_act(s, cm, p["bn2"]["g"], p["bn2"]["b"], residual=x_p, relu=True)
    return o


def _prep_edges(srcs, dsts):
    src_p = jnp.pad(srcs, ((0, 0), (0, E_PAD_K - E_PER_K)))
    padvals = (N_NODES + (jnp.arange(E_PAD_K - E_PER_K) % (N_PAD - N_NODES)))
    dst_p = jnp.concatenate(
        [dsts, jnp.broadcast_to(padvals.astype(jnp.int32),
                                (K_VOL, E_PAD_K - E_PER_K))], axis=1)
    return src_p.reshape(-1), dst_p.reshape(-1)


def kernel(x, edge_index, params):
    src_f, dst_f = _prep_edges(edge_index[:, 0, :], edge_index[:, 1, :])
    src_t, dst_t = _prep_edges(edge_index[:, 1, :], edge_index[:, 0, :])

    p = params
    xp = jnp.pad(x, ((0, 0), (0, 125)))

    def conv_bn(h_p, segs, w, nrm, transposed=False):
        s = src_t if transposed else src_f
        d = dst_t if transposed else dst_f
        sp, cm = _sconv(h_p, _pad_w(w, segs), s, d)
        return _bn_act(sp, cm, nrm["g"], nrm["b"], relu=False)

    def blk(h_p, c, bp):
        return _block(h_p, c, bp, src_f, dst_f)

    s1 = blk(conv_bn(xp, [(3, 128)], p["conv1"], p["norm1"]), 32, p["block1"])
    s2 = blk(conv_bn(s1, [(32, 128)], p["conv2"], p["norm2"]), 64, p["block2"])
    s4 = blk(conv_bn(s2, [(64, 128)], p["conv3"], p["norm3"]), 128, p["block3"])
    s8 = blk(conv_bn(s4, [(128, 128)], p["conv4"], p["norm4"]), 256, p["block4"])

    out = blk(conv_bn(s8, [(256, 256)], p["conv4_tr"], p["norm4_tr"], True),
              128, p["block4_tr"])
    out = jnp.concatenate([out, s4], axis=1)
    out = blk(conv_bn(out, [(128, 128), (128, 128)], p["conv3_tr"],
                      p["norm3_tr"], True), 64, p["block3_tr"])
    out = jnp.concatenate([out, s2], axis=1)
    out = blk(conv_bn(out, [(64, 128), (64, 128)], p["conv2_tr"],
                      p["norm2_tr"], True), 64, p["block2_tr"])
    out = jnp.concatenate([out, s1], axis=1)

    sp, _ = _sconv(out, _pad_w(p["conv1_tr"], [(64, 128), (32, 128)]),
                   src_t, dst_t)
    return _final(sp, p["final_w"], p["final_b"])


# speedup vs baseline: 10.2294x; 2.0335x over previous
import functools

import jax
import jax.numpy as jnp
from jax import lax
from jax.experimental import pallas as pl
from jax.experimental.pallas import tpu as pltpu
from jax.experimental.pallas import tpu_sc as plsc

N_NODES = 10000
K_VOL = 27
E_PER_K = 5000

_NC = 2
_NS = 16
_NW = _NC * _NS
E_PAD_K = 5120
E_PAD = K_VOL * E_PAD_K
N_PAD = 10240
Z_ROWS = K_VOL * N_PAD
_RG = 120


def _ceil128(c):
    return ((c + 127) // 128) * 128


@functools.partial(jax.jit, static_argnames=("split_ch",))
def _sc_gs(z, src_adj, dst, split_ch):
    cout = 128
    per_sub_rows = N_PAD // _NS
    per_edge = (E_PAD if split_ch else E_PAD // 2) // _NS
    chunks = per_edge // _RG
    zeros = jnp.zeros((N_PAD, cout), jnp.float32)
    mesh = plsc.VectorSubcoreMesh(core_axis_name="c", subcore_axis_name="s")

    @functools.partial(
        pl.kernel,
        mesh=mesh,
        out_type=jax.ShapeDtypeStruct((2, N_PAD, cout), jnp.float32),
        scratch_types=[
            pltpu.VMEM((_RG,), jnp.int32),
            pltpu.VMEM((_RG,), jnp.int32),
            pltpu.VMEM((_RG, cout), jnp.float32),
            pltpu.VMEM_SHARED((N_PAD, cout), jnp.float32),
        ],
    )
    def k(z_hbm, src_hbm, dst_hbm, zz_hbm, out_hbm, sidx_v, didx_v, rows_v,
          acc):
        c = lax.axis_index("c")
        s = lax.axis_index("s")
        row0 = s * per_sub_rows
        pltpu.sync_copy(zz_hbm.at[pl.ds(row0, per_sub_rows)],
                        acc.at[pl.ds(row0, per_sub_rows)])
        plsc.subcore_barrier()
        if split_ch:
            ebase = s * per_edge
            sbase = c * E_PAD + ebase
        else:
            ebase = c * (E_PAD // 2) + s * per_edge
            sbase = ebase

        def body(i, carry):
            eoff = ebase + i * _RG
            pltpu.sync_copy(src_hbm.at[pl.ds(sbase + i * _RG, _RG)], sidx_v)
            pltpu.sync_copy(dst_hbm.at[pl.ds(eoff, _RG)], didx_v)
            pltpu.sync_copy(z_hbm.at[sidx_v], rows_v)
            pltpu.sync_copy(rows_v, acc.at[didx_v], add=True)
            return carry

        lax.fori_loop(0, chunks, body, 0)
        plsc.subcore_barrier()
        pltpu.sync_copy(acc.at[pl.ds(row0, per_sub_rows)],
                        out_hbm.at[c, pl.ds(row0, per_sub_rows)])

    return k(z, src_adj, dst, zeros)


def _dense_mm_body(x_ref, w_ref, o_ref):
    o_ref[...] = jnp.dot(x_ref[...], w_ref[0],
                         preferred_element_type=jnp.float32)[None]


def _dense_mm(x, w):
    cp = x.shape[1]
    kk, _, cout = w.shape
    if cout < 128:
        w = jnp.concatenate(
            [w, jnp.zeros((kk, cp, 128 - cout), w.dtype)], axis=2)
    out = pl.pallas_call(
        _dense_mm_body,
        grid=(kk,),
        in_specs=[
            pl.BlockSpec((N_PAD, cp), lambda i: (0, 0)),
            pl.BlockSpec((1, cp, 128), lambda i: (i, 0, 0)),
        ],
        out_specs=pl.BlockSpec((1, N_PAD, 128), lambda i: (i, 0, 0)),
        out_shape=jax.ShapeDtypeStruct((kk, N_PAD, 128), jnp.float32),
    )(x, w)
    return out.reshape(Z_ROWS, 128)


def _dense_mm_split_body(x_ref, w_ref, o_ref):
    o_ref[...] = jnp.dot(x_ref[...], w_ref[0],
                         preferred_element_type=jnp.float32)[None, None]


def _dense_mm_split(x, w):
    cp = x.shape[1]
    kk = w.shape[0]
    out = pl.pallas_call(
        _dense_mm_split_body,
        grid=(kk, 2),
        in_specs=[
            pl.BlockSpec((N_PAD, cp), lambda i, h: (0, 0)),
            pl.BlockSpec((1, cp, 128), lambda i, h: (i, 0, h)),
        ],
        out_specs=pl.BlockSpec((1, 1, N_PAD, 128), lambda i, h: (h, i, 0, 0)),
        out_shape=jax.ShapeDtypeStruct((2, kk, N_PAD, 128), jnp.float32),
    )(x, w)
    return out.reshape(2 * Z_ROWS, 128)


def _bn_body(relu, has_res, cp, concat_mode, *refs):
    if has_res:
        s_ref, g_ref, b_ref, r_ref, o_ref = refs
    else:
        s_ref, g_ref, b_ref, o_ref = refs
    if concat_mode:
        x = jnp.concatenate([s_ref[0, :N_NODES, :], s_ref[1, :N_NODES, :]],
                            axis=1)
    else:
        x = s_ref[0, :N_NODES, :] + s_ref[1, :N_NODES, :]
    c = g_ref.shape[1]
    x = x[:, :c]
    m = jnp.mean(x, axis=0, keepdims=True)
    v = jnp.mean((x - m) * (x - m), axis=0, keepdims=True)
    y = (x - m) * jax.lax.rsqrt(v + 1e-5) * g_ref[...] + b_ref[...]
    if cp > c:
        y = jnp.concatenate([y, jnp.zeros((N_NODES, cp - c), y.dtype)],
                            axis=1)
    y = jnp.concatenate([y, jnp.zeros((N_PAD - N_NODES, cp), y.dtype)],
                        axis=0)
    if has_res:
        y = y + r_ref[...]
    if relu:
        y = jnp.maximum(y, 0.0)
    o_ref[...] = y


def _bn_act(s_pair, concat_mode, g, b, residual=None, relu=True):
    c = g.shape[0]
    cp = _ceil128(c)
    args = [s_pair, g.reshape(1, c), b.reshape(1, c)]
    if residual is not None:
        args.append(residual)
    return pl.pallas_call(
        functools.partial(_bn_body, relu, residual is not None, cp,
                          concat_mode),
        out_shape=jax.ShapeDtypeStruct((N_PAD, cp), jnp.float32),
    )(*args)


def _final_body(x_ref, w_ref, b_ref, o_ref):
    cin = w_ref.shape[0]
    x = (x_ref[0, :N_NODES, :] + x_ref[1, :N_NODES, :])[:, :cin]
    y = jnp.dot(jnp.maximum(x, 0.0), w_ref[...],
                preferred_element_type=jnp.float32) + b_ref[...]
    nrm = jnp.sqrt(jnp.sum(y * y, axis=1, keepdims=True))
    o_ref[...] = y / (nrm + 1e-8)


def _final(x_pair, w, b):
    cout = w.shape[1]
    return pl.pallas_call(
        _final_body,
        out_shape=jax.ShapeDtypeStruct((N_NODES, cout), jnp.float32),
    )(x_pair, w.astype(jnp.float32), b.reshape(1, cout))


def _pad_w(w, segs):
    parts, o = [], 0
    for real, padw in segs:
        parts.append(w[:, o:o + real, :])
        if padw > real:
            parts.append(jnp.zeros((K_VOL, padw - real, w.shape[2]), w.dtype))
        o += real
    if len(parts) == 1:
        return parts[0]
    return jnp.concatenate(parts, axis=1)


def _sconv(x_p, w_p, src_pair, dst_flat):
    cout = w_p.shape[2]
    if cout == 256:
        z = _dense_mm_split(x_p, w_p)
        return _sc_gs(z, src_pair[1], dst_flat, True), True
    z = _dense_mm(x_p, w_p)
    return _sc_gs(z, src_pair[0], dst_flat, False), False


def _block(x_p, c, p, src, dst):
    cp = x_p.shape[1]
    segs = [(c, cp)]
    s, cm = _sconv(x_p, _pad_w(p["w1"], segs), src, dst)
    o = _bn_act(s, cm, p["bn1"]["g"], p["bn1"]["b"], relu=True)
    s, cm = _sconv(o, _pad_w(p["w2"], segs), src, dst)
    o = _bn_act(s, cm, p["bn2"]["g"], p["bn2"]["b"], residual=x_p, relu=True)
    return o


def _prep_edges(srcs, dsts):
    src_p = jnp.pad(srcs, ((0, 0), (0, E_PAD_K - E_PER_K)))
    src_adj = (src_p.astype(jnp.int32)
               + (jnp.arange(K_VOL, dtype=jnp.int32) * N_PAD)[:, None]
               ).reshape(-1)
    src_both = jnp.concatenate([src_adj, src_adj + Z_ROWS])
    padvals = (N_NODES + (jnp.arange(E_PAD_K - E_PER_K) % (N_PAD - N_NODES)))
    dst_p = jnp.concatenate(
        [dsts, jnp.broadcast_to(padvals.astype(jnp.int32),
                                (K_VOL, E_PAD_K - E_PER_K))], axis=1)
    return (src_adj, src_both), dst_p.reshape(-1)


def kernel(x, edge_index, params):
    src_f, dst_f = _prep_edges(edge_index[:, 0, :], edge_index[:, 1, :])
    src_t, dst_t = _prep_edges(edge_index[:, 1, :], edge_index[:, 0, :])

    p = params
    xp = jnp.pad(x, ((0, N_PAD - N_NODES), (0, 125)))

    def conv_bn(h_p, segs, w, nrm, transposed=False):
        s = src_t if transposed else src_f
        d = dst_t if transposed else dst_f
        sp, cm = _sconv(h_p, _pad_w(w, segs), s, d)
        return _bn_act(sp, cm, nrm["g"], nrm["b"], relu=False)

    def blk(h_p, c, bp):
        return _block(h_p, c, bp, src_f, dst_f)

    s1 = blk(conv_bn(xp, [(3, 128)], p["conv1"], p["norm1"]), 32, p["block1"])
    s2 = blk(conv_bn(s1, [(32, 128)], p["conv2"], p["norm2"]), 64, p["block2"])
    s4 = blk(conv_bn(s2, [(64, 128)], p["conv3"], p["norm3"]), 128, p["block3"])
    s8 = blk(conv_bn(s4, [(128, 128)], p["conv4"], p["norm4"]), 256, p["block4"])

    out = blk(conv_bn(s8, [(256, 256)], p["conv4_tr"], p["norm4_tr"], True),
              128, p["block4_tr"])
    out = jnp.concatenate([out, s4], axis=1)
    out = blk(conv_bn(out, [(128, 128), (128, 128)], p["conv3_tr"],
                      p["norm3_tr"], True), 64, p["block3_tr"])
    out = jnp.concatenate([out, s2], axis=1)
    out = blk(conv_bn(out, [(64, 128), (64, 128)], p["conv2_tr"],
                      p["norm2_tr"], True), 64, p["block2_tr"])
    out = jnp.concatenate([out, s1], axis=1)

    sp, _ = _sconv(out, _pad_w(p["conv1_tr"], [(64, 128), (32, 128)]),
                   src_t, dst_t)
    return _final(sp, p["final_w"], p["final_b"])


# speedup vs baseline: 13.7217x; 1.3414x over previous
import functools

import jax
import jax.numpy as jnp
from jax import lax
from jax.experimental import pallas as pl
from jax.experimental.pallas import tpu as pltpu
from jax.experimental.pallas import tpu_sc as plsc

N_NODES = 10000
K_VOL = 27
E_PER_K = 5000

_NC = 2
_NS = 16
_NW = _NC * _NS
E_PAD_K = 5120
E_PAD = K_VOL * E_PAD_K
N_PAD = 10240
Z_ROWS = K_VOL * N_PAD
_RG = 120


def _ceil128(c):
    return ((c + 127) // 128) * 128


@functools.partial(jax.jit, static_argnames=("split_ch",))
def _sc_gs(z, src_adj, dst, split_ch):
    cout = 128
    per_sub_rows = N_PAD // _NS
    per_edge = (E_PAD if split_ch else E_PAD // 2) // _NS
    chunks = per_edge // _RG
    zeros = jnp.zeros((N_PAD, cout), jnp.float32)
    mesh = plsc.VectorSubcoreMesh(core_axis_name="c", subcore_axis_name="s")

    pairs = chunks // 2

    @functools.partial(
        pl.kernel,
        mesh=mesh,
        out_type=jax.ShapeDtypeStruct((2, N_PAD, cout), jnp.float32),
        scratch_types=[
            pltpu.VMEM((_RG,), jnp.int32),
            pltpu.VMEM((_RG,), jnp.int32),
            pltpu.VMEM((_RG,), jnp.int32),
            pltpu.VMEM((_RG,), jnp.int32),
            pltpu.VMEM((_RG, cout), jnp.float32),
            pltpu.VMEM((_RG, cout), jnp.float32),
            pltpu.VMEM_SHARED((N_PAD, cout), jnp.float32),
            pltpu.SemaphoreType.DMA,
            pltpu.SemaphoreType.DMA,
            pltpu.SemaphoreType.DMA,
            pltpu.SemaphoreType.DMA,
            pltpu.SemaphoreType.DMA,
            pltpu.SemaphoreType.DMA,
        ],
    )
    def k(z_hbm, src_hbm, dst_hbm, zz_hbm, out_hbm,
          sidx0, sidx1, didx0, didx1, rows0, rows1, acc,
          ss0, ss1, ds0, ds1, g0, g1):
        c = lax.axis_index("c")
        s = lax.axis_index("s")
        row0 = s * per_sub_rows
        pltpu.sync_copy(zz_hbm.at[pl.ds(row0, per_sub_rows)],
                        acc.at[pl.ds(row0, per_sub_rows)])
        plsc.subcore_barrier()
        if split_ch:
            ebase = s * per_edge
            sbase = c * E_PAD + ebase
        else:
            ebase = c * (E_PAD // 2) + s * per_edge
            sbase = ebase

        def idx_start(i, sb, db, ssm, dsm):
            pltpu.async_copy(src_hbm.at[pl.ds(sbase + i * _RG, _RG)], sb, ssm)
            pltpu.async_copy(dst_hbm.at[pl.ds(ebase + i * _RG, _RG)], db, dsm)

        def idx_wait(sb, db, ssm, dsm):
            pltpu.make_async_copy(src_hbm.at[pl.ds(sbase, _RG)], sb,
                                  ssm).wait()
            pltpu.make_async_copy(dst_hbm.at[pl.ds(ebase, _RG)], db,
                                  dsm).wait()

        def g_start(sb, rb, gsm):
            pltpu.async_copy(z_hbm.at[sb], rb, gsm)

        def g_wait(sb, rb, gsm):
            pltpu.make_async_copy(z_hbm.at[sb], rb, gsm).wait()

        idx_start(0, sidx0, didx0, ss0, ds0)
        idx_wait(sidx0, didx0, ss0, ds0)
        g_start(sidx0, rows0, g0)
        idx_start(1, sidx1, didx1, ss1, ds1)

        def body(j, carry):
            i = 2 * j
            idx_wait(sidx1, didx1, ss1, ds1)
            g_start(sidx1, rows1, g1)
            g_wait(sidx0, rows0, g0)
            pltpu.sync_copy(rows0, acc.at[didx0], add=True)

            @pl.when(j < pairs - 1)
            def _():
                idx_start(i + 2, sidx0, didx0, ss0, ds0)

            g_wait(sidx1, rows1, g1)
            pltpu.sync_copy(rows1, acc.at[didx1], add=True)

            @pl.when(j < pairs - 1)
            def _():
                idx_wait(sidx0, didx0, ss0, ds0)
                g_start(sidx0, rows0, g0)
                idx_start(i + 3, sidx1, didx1, ss1, ds1)

            return carry

        lax.fori_loop(0, pairs, body, 0)
        plsc.subcore_barrier()
        pltpu.sync_copy(acc.at[pl.ds(row0, per_sub_rows)],
                        out_hbm.at[c, pl.ds(row0, per_sub_rows)])

    return k(z, src_adj, dst, zeros)


def _dense_mm_body(x_ref, w_ref, o_ref):
    o_ref[...] = jnp.dot(x_ref[...], w_ref[0],
                         preferred_element_type=jnp.float32)[None]


def _dense_mm(x, w):
    cp = x.shape[1]
    kk, _, cout = w.shape
    if cout < 128:
        w = jnp.concatenate(
            [w, jnp.zeros((kk, cp, 128 - cout), w.dtype)], axis=2)
    out = pl.pallas_call(
        _dense_mm_body,
        grid=(kk,),
        in_specs=[
            pl.BlockSpec((N_PAD, cp), lambda i: (0, 0)),
            pl.BlockSpec((1, cp, 128), lambda i: (i, 0, 0)),
        ],
        out_specs=pl.BlockSpec((1, N_PAD, 128), lambda i: (i, 0, 0)),
        out_shape=jax.ShapeDtypeStruct((kk, N_PAD, 128), jnp.float32),
    )(x, w)
    return out.reshape(Z_ROWS, 128)


def _dense_mm_split_body(x_ref, w_ref, o_ref):
    o_ref[...] = jnp.dot(x_ref[...], w_ref[0],
                         preferred_element_type=jnp.float32)[None, None]


def _dense_mm_split(x, w):
    cp = x.shape[1]
    kk = w.shape[0]
    out = pl.pallas_call(
        _dense_mm_split_body,
        grid=(kk, 2),
        in_specs=[
            pl.BlockSpec((N_PAD, cp), lambda i, h: (0, 0)),
            pl.BlockSpec((1, cp, 128), lambda i, h: (i, 0, h)),
        ],
        out_specs=pl.BlockSpec((1, 1, N_PAD, 128), lambda i, h: (h, i, 0, 0)),
        out_shape=jax.ShapeDtypeStruct((2, kk, N_PAD, 128), jnp.float32),
    )(x, w)
    return out.reshape(2 * Z_ROWS, 128)


def _bn_body(relu, has_res, cp, concat_mode, *refs):
    if has_res:
        s_ref, g_ref, b_ref, r_ref, o_ref = refs
    else:
        s_ref, g_ref, b_ref, o_ref = refs
    if concat_mode:
        x = jnp.concatenate([s_ref[0, :N_NODES, :], s_ref[1, :N_NODES, :]],
                            axis=1)
    else:
        x = s_ref[0, :N_NODES, :] + s_ref[1, :N_NODES, :]
    c = g_ref.shape[1]
    x = x[:, :c]
    m = jnp.mean(x, axis=0, keepdims=True)
    v = jnp.mean((x - m) * (x - m), axis=0, keepdims=True)
    y = (x - m) * jax.lax.rsqrt(v + 1e-5) * g_ref[...] + b_ref[...]
    if cp > c:
        y = jnp.concatenate([y, jnp.zeros((N_NODES, cp - c), y.dtype)],
                            axis=1)
    y = jnp.concatenate([y, jnp.zeros((N_PAD - N_NODES, cp), y.dtype)],
                        axis=0)
    if has_res:
        y = y + r_ref[...]
    if relu:
        y = jnp.maximum(y, 0.0)
    o_ref[...] = y


def _bn_act(s_pair, concat_mode, g, b, residual=None, relu=True):
    c = g.shape[0]
    cp = _ceil128(c)
    args = [s_pair, g.reshape(1, c), b.reshape(1, c)]
    if residual is not None:
        args.append(residual)
    return pl.pallas_call(
        functools.partial(_bn_body, relu, residual is not None, cp,
                          concat_mode),
        out_shape=jax.ShapeDtypeStruct((N_PAD, cp), jnp.float32),
    )(*args)


def _final_body(x_ref, w_ref, b_ref, o_ref):
    cin = w_ref.shape[0]
    x = (x_ref[0, :N_NODES, :] + x_ref[1, :N_NODES, :])[:, :cin]
    y = jnp.dot(jnp.maximum(x, 0.0), w_ref[...],
                preferred_element_type=jnp.float32) + b_ref[...]
    nrm = jnp.sqrt(jnp.sum(y * y, axis=1, keepdims=True))
    o_ref[...] = y / (nrm + 1e-8)


def _final(x_pair, w, b):
    cout = w.shape[1]
    return pl.pallas_call(
        _final_body,
        out_shape=jax.ShapeDtypeStruct((N_NODES, cout), jnp.float32),
    )(x_pair, w.astype(jnp.float32), b.reshape(1, cout))


def _pad_w(w, segs):
    parts, o = [], 0
    for real, padw in segs:
        parts.append(w[:, o:o + real, :])
        if padw > real:
            parts.append(jnp.zeros((K_VOL, padw - real, w.shape[2]), w.dtype))
        o += real
    if len(parts) == 1:
        return parts[0]
    return jnp.concatenate(parts, axis=1)


def _sconv(x_p, w_p, src_pair, dst_flat):
    cout = w_p.shape[2]
    if cout == 256:
        z = _dense_mm_split(x_p, w_p)
        return _sc_gs(z, src_pair[1], dst_flat, True), True
    z = _dense_mm(x_p, w_p)
    return _sc_gs(z, src_pair[0], dst_flat, False), False


def _block(x_p, c, p, src, dst):
    cp = x_p.shape[1]
    segs = [(c, cp)]
    s, cm = _sconv(x_p, _pad_w(p["w1"], segs), src, dst)
    o = _bn_act(s, cm, p["bn1"]["g"], p["bn1"]["b"], relu=True)
    s, cm = _sconv(o, _pad_w(p["w2"], segs), src, dst)
    o = _bn_act(s, cm, p["bn2"]["g"], p["bn2"]["b"], residual=x_p, relu=True)
    return o


def _prep_edges(srcs, dsts):
    src_p = jnp.pad(srcs, ((0, 0), (0, E_PAD_K - E_PER_K)))
    src_adj = (src_p.astype(jnp.int32)
               + (jnp.arange(K_VOL, dtype=jnp.int32) * N_PAD)[:, None]
               ).reshape(-1)
    src_both = jnp.concatenate([src_adj, src_adj + Z_ROWS])
    padvals = (N_NODES + (jnp.arange(E_PAD_K - E_PER_K) % (N_PAD - N_NODES)))
    dst_p = jnp.concatenate(
        [dsts, jnp.broadcast_to(padvals.astype(jnp.int32),
                                (K_VOL, E_PAD_K - E_PER_K))], axis=1)
    return (src_adj, src_both), dst_p.reshape(-1)


def kernel(x, edge_index, params):
    src_f, dst_f = _prep_edges(edge_index[:, 0, :], edge_index[:, 1, :])
    src_t, dst_t = _prep_edges(edge_index[:, 1, :], edge_index[:, 0, :])

    p = params
    xp = jnp.pad(x, ((0, N_PAD - N_NODES), (0, 125)))

    def conv_bn(h_p, segs, w, nrm, transposed=False):
        s = src_t if transposed else src_f
        d = dst_t if transposed else dst_f
        sp, cm = _sconv(h_p, _pad_w(w, segs), s, d)
        return _bn_act(sp, cm, nrm["g"], nrm["b"], relu=False)

    def blk(h_p, c, bp):
        return _block(h_p, c, bp, src_f, dst_f)

    s1 = blk(conv_bn(xp, [(3, 128)], p["conv1"], p["norm1"]), 32, p["block1"])
    s2 = blk(conv_bn(s1, [(32, 128)], p["conv2"], p["norm2"]), 64, p["block2"])
    s4 = blk(conv_bn(s2, [(64, 128)], p["conv3"], p["norm3"]), 128, p["block3"])
    s8 = blk(conv_bn(s4, [(128, 128)], p["conv4"], p["norm4"]), 256, p["block4"])

    out = blk(conv_bn(s8, [(256, 256)], p["conv4_tr"], p["norm4_tr"], True),
              128, p["block4_tr"])
    out = jnp.concatenate([out, s4], axis=1)
    out = blk(conv_bn(out, [(128, 128), (128, 128)], p["conv3_tr"],
                      p["norm3_tr"], True), 64, p["block3_tr"])
    out = jnp.concatenate([out, s2], axis=1)
    out = blk(conv_bn(out, [(64, 128), (64, 128)], p["conv2_tr"],
                      p["norm2_tr"], True), 64, p["block2_tr"])
    out = jnp.concatenate([out, s1], axis=1)

    sp, _ = _sconv(out, _pad_w(p["conv1_tr"], [(64, 128), (32, 128)]),
                   src_t, dst_t)
    return _final(sp, p["final_w"], p["final_b"])
